# baseline (speedup 1.0000x reference)
import functools

import jax
import jax.numpy as jnp
from jax import lax
from jax.experimental import pallas as pl
from jax.experimental.pallas import tpu as pltpu
from jax.experimental.pallas import tpu_sc as plsc

_N_USERS = 25000
_N_NODES = 50000
_E = 800000
_D = 64
_H = 32
_NC = 2
_NS = 16
_LANES = 16

_EPAD = 802816
_ROWS128 = _EPAD // 128
_TILE_E = _EPAD // _NS
_CHUNK = 256
_GPC = _CHUNK // 128
_NCHUNK = _TILE_E // _CHUNK
_NB = 3
_ACC_ROWS = 50176
_ZSTRIPE = _ACC_ROWS // _NS
_OSTRIPE = 3128


def _sc_spmm(ego_stack, meta0, meta1, val2):
    mesh = plsc.VectorSubcoreMesh(
        core_axis_name="c", subcore_axis_name="s",
        num_cores=_NC, num_subcores=_NS)

    @functools.partial(
        pl.kernel,
        out_type=jax.ShapeDtypeStruct((_NC * _N_NODES, _H), jnp.float32),
        mesh=mesh,
        scratch_types=[
            pltpu.VMEM((_NB, 2, _CHUNK), jnp.int32),
            pltpu.VMEM((_NB * _CHUNK,), jnp.float32),
            pltpu.VMEM((_NB * _CHUNK, _H), jnp.float32),
            pltpu.VMEM_SHARED((_ACC_ROWS, _H), jnp.float32),
            pltpu.SemaphoreType.DMA((2,)),
            pltpu.SemaphoreType.DMA((2,)),
            pltpu.SemaphoreType.DMA,
        ],
        compiler_params=pltpu.CompilerParams(use_tc_tiling_on_sc=False),
    )
    def spmm_kernel(ego_hbm, meta0_hbm, meta1_hbm, val_hbm, out_hbm,
                    meta_buf, val_buf, rows_buf, acc, sem_i, sem_g, sem_s):
        c = lax.axis_index("c")
        s = lax.axis_index("s")

        def fire_idx(g, k):
            cid = s * _NCHUNK + g

            @pl.when(c == 0)
            def _f0():
                pltpu.async_copy(
                    meta0_hbm.at[cid], meta_buf.at[k], sem_i.at[g % 2])

            @pl.when(c == 1)
            def _f1():
                pltpu.async_copy(
                    meta1_hbm.at[cid], meta_buf.at[k], sem_i.at[g % 2])

            pltpu.async_copy(
                val_hbm.at[cid], val_buf.at[pl.ds(k * _CHUNK, _CHUNK)],
                sem_i.at[g % 2])

        def wait_idx(g, k):
            pltpu.make_async_copy(
                meta0_hbm.at[0], meta_buf.at[k], sem_i.at[g % 2]).wait()
            pltpu.make_async_copy(
                val_hbm.at[0], val_buf.at[pl.ds(k * _CHUNK, _CHUNK)],
                sem_i.at[g % 2]).wait()

        def fire_gather(g, k):
            pltpu.async_copy(
                ego_hbm.at[meta_buf.at[k, 0]],
                rows_buf.at[pl.ds(k * _CHUNK, _CHUNK)], sem_g.at[g % 2])

        def wait_gather(g, k):
            pltpu.make_async_copy(
                ego_hbm.at[meta_buf.at[k, 0]],
                rows_buf.at[pl.ds(k * _CHUNK, _CHUNK)], sem_g.at[g % 2]).wait()

        def fire_scatter(k):
            pltpu.async_copy(
                rows_buf.at[pl.ds(k * _CHUNK, _CHUNK)],
                acc.at[meta_buf.at[k, 1]], sem_s, add=True)

        def wait_scatter(k):
            pltpu.make_async_copy(
                rows_buf.at[pl.ds(k * _CHUNK, _CHUNK)],
                acc.at[meta_buf.at[k, 1]], sem_s).wait()

        zro = jnp.zeros((_LANES,), jnp.float32)

        def zero_body(i, _):
            rows_buf[i // 2, pl.ds((i % 2) * _LANES, _LANES)] = zro
            return ()
        lax.fori_loop(0, _NB * _CHUNK * 2, zero_body, (), unroll=8)

        zrows = _NB * _CHUNK
        nfull = _ZSTRIPE // zrows
        zrem = _ZSTRIPE % zrows
        for j in range(nfull):
            pltpu.sync_copy(
                rows_buf, acc.at[pl.ds(s * _ZSTRIPE + j * zrows, zrows)])
        if zrem:
            pltpu.sync_copy(
                rows_buf.at[pl.ds(0, zrem)],
                acc.at[pl.ds(s * _ZSTRIPE + nfull * zrows, zrem)])
        plsc.subcore_barrier()

        fire_idx(0, 0)
        fire_idx(1, 1)
        wait_idx(0, 0)
        fire_gather(0, 0)

        def chunk_body(g, _):
            k = g % _NB
            kn = (g + 1) % _NB
            kn2 = (g + 2) % _NB

            @pl.when(g < _NCHUNK - 1)
            def _advance():
                wait_idx(g + 1, kn)
                fire_gather(g + 1, kn)

            wait_gather(g, k)

            dnums = lax.GatherDimensionNumbers(
                offset_dims=(), collapsed_slice_dims=(0,),
                start_index_map=(0,))

            def mul_body(b, _):
                e0 = b * _LANES
                vv = val_buf[pl.ds(k * _CHUNK + e0, _LANES)]
                for l in range(_LANES):
                    bv = lax.gather(
                        vv, jnp.zeros((_LANES, 1), jnp.int32) + l, dnums,
                        (1,), mode=lax.GatherScatterMode.PROMISE_IN_BOUNDS)
                    r = k * _CHUNK + e0 + l
                    rows_buf[r, pl.ds(0, _LANES)] = (
                        rows_buf[r, pl.ds(0, _LANES)] * bv)
                    rows_buf[r, pl.ds(_LANES, _LANES)] = (
                        rows_buf[r, pl.ds(_LANES, _LANES)] * bv)
                return ()
            lax.fori_loop(0, _CHUNK // _LANES, mul_body, (), unroll=2)

            @pl.when(g >= 1)
            def _drain_scatter():
                wait_scatter(kn2)

            @pl.when(g < _NCHUNK - 2)
            def _prefetch():
                fire_idx(g + 2, kn2)

            fire_scatter(k)
            return ()
        lax.fori_loop(0, _NCHUNK, chunk_body, ())
        wait_scatter((_NCHUNK - 1) % _NB)

        plsc.subcore_barrier()
        last = _N_NODES - (_NS - 1) * _OSTRIPE

        @pl.when(s < _NS - 1)
        def _copy_main():
            pltpu.sync_copy(
                acc.at[pl.ds(s * _OSTRIPE, _OSTRIPE)],
                out_hbm.at[pl.ds(c * _N_NODES + s * _OSTRIPE, _OSTRIPE)])

        @pl.when(s == _NS - 1)
        def _copy_last():
            pltpu.sync_copy(
                acc.at[pl.ds((_NS - 1) * _OSTRIPE, last)],
                out_hbm.at[pl.ds(c * _N_NODES + (_NS - 1) * _OSTRIPE, last)])

    return spmm_kernel(ego_stack, meta0, meta1, val2)


_R = 2000
_G = _N_NODES // _R


def _tc_dense(side_stack, ego_stack, acc, w_gc, b_gc, w_bi, b_bi, scale):
    def body(sl, sr, el, er, acc_in, wgc, bgc, wbi, bbi, ego_out, acc_out):
        s_full = jnp.concatenate([sl[...], sr[...]], axis=1)
        e_full = jnp.concatenate([el[...], er[...]], axis=1)
        sum_e = jnp.dot(s_full, wgc[...],
                        preferred_element_type=jnp.float32) + bgc[...]
        bi = jnp.dot(e_full * s_full, wbi[...],
                     preferred_element_type=jnp.float32) + bbi[...]
        x = sum_e + bi
        x = jnp.where(x >= 0, x, 0.2 * x)
        n = jnp.sqrt(jnp.sum(x * x, axis=1, keepdims=True))
        xn = x / jnp.maximum(n, 1e-12)
        acc_out[...] = (acc_in[...] + xn) * scale
        ego_out[0] = x[:, :_H]
        ego_out[1] = x[:, _H:]

    half = lambda off: pl.BlockSpec((_R, _H), lambda i: (i + off, 0))
    full_rows = pl.BlockSpec((_R, _D), lambda i: (i, 0))
    wspec = pl.BlockSpec((_D, _D), lambda i: (0, 0))
    bspec = pl.BlockSpec((1, _D), lambda i: (0, 0))
    ego3, acc_new = pl.pallas_call(
        body,
        grid=(_G,),
        in_specs=[half(0), half(_G), half(0), half(_G), full_rows,
                  wspec, bspec, wspec, bspec],
        out_specs=[pl.BlockSpec((2, _R, _H), lambda i: (0, i, 0)), full_rows],
        out_shape=[
            jax.ShapeDtypeStruct((2, _N_NODES, _H), jnp.float32),
            jax.ShapeDtypeStruct((_N_NODES, _D), jnp.float32),
        ],
    )(side_stack, side_stack, ego_stack, ego_stack, acc,
      w_gc, b_gc, w_bi, b_bi)
    return ego3, acc_new


def kernel(user_emb, item_emb, adj_val,
           W_gc_0, b_gc_0, W_bi_0, b_bi_0,
           W_gc_1, b_gc_1, W_bi_1, b_bi_1,
           W_gc_2, b_gc_2, W_bi_2, b_bi_2,
           adj_row, adj_col):
    ws_gc = [W_gc_0, W_gc_1, W_gc_2]
    bs_gc = [b_gc_0, b_gc_1, b_gc_2]
    ws_bi = [W_bi_0, W_bi_1, W_bi_2]
    bs_bi = [b_bi_0, b_bi_1, b_bi_2]

    ego_l = jnp.concatenate([user_emb[:, :_H], item_emb[:, :_H]], axis=0)
    ego_r = jnp.concatenate([user_emb[:, _H:], item_emb[:, _H:]], axis=0)
    ego_stack = jnp.concatenate([ego_l, ego_r], axis=0)

    pad = _EPAD - _E
    colp = jnp.concatenate([adj_col, jnp.zeros((pad,), jnp.int32)])
    rowp = jnp.concatenate([adj_row, jnp.zeros((pad,), jnp.int32)])
    ncid = _NS * _NCHUNK
    val2 = jnp.concatenate(
        [adj_val, jnp.zeros((pad,), jnp.float32)]).reshape(ncid, _CHUNK)
    meta0 = jnp.stack(
        [colp.reshape(ncid, _CHUNK), rowp.reshape(ncid, _CHUNK)], axis=1)
    meta1 = meta0.at[:, 0, :].add(_N_NODES)

    acc = jnp.concatenate([user_emb, item_emb], axis=0)
    n_layers = 3
    for k in range(n_layers):
        side_stack = _sc_spmm(ego_stack, meta0, meta1, val2)
        scale = 0.25 if k == n_layers - 1 else 1.0
        ego3, acc = _tc_dense(side_stack, ego_stack, acc,
                              ws_gc[k], bs_gc[k], ws_bi[k], bs_bi[k], scale)
        ego_stack = ego3.reshape(_NC * _N_NODES, _H)

    return acc[:_N_USERS], acc[_N_USERS:]


# speedup vs baseline: 1.7450x; 1.7450x over previous
import functools

import jax
import jax.numpy as jnp
from jax import lax
from jax.experimental import pallas as pl
from jax.experimental.pallas import tpu as pltpu
from jax.experimental.pallas import tpu_sc as plsc

_N_USERS = 25000
_N_NODES = 50000
_E = 800000
_D = 64
_H = 32
_NC = 2
_NS = 16
_LANES = 16

_EPAD = 802816
_ROWS128 = _EPAD // 128
_TILE_E = _EPAD // _NS
_CHUNK = 256
_GPC = _CHUNK // 128
_NCHUNK = _TILE_E // _CHUNK
_NB = 3
_ACC_ROWS = 50176
_ZSTRIPE = _ACC_ROWS // _NS
_OSTRIPE = 3128


def _sc_spmm(ego_stack, meta0, meta1, val2):
    mesh = plsc.VectorSubcoreMesh(
        core_axis_name="c", subcore_axis_name="s",
        num_cores=_NC, num_subcores=_NS)

    @functools.partial(
        pl.kernel,
        out_type=jax.ShapeDtypeStruct((_NC * _N_NODES, _H), jnp.float32),
        mesh=mesh,
        scratch_types=[
            pltpu.VMEM((_NB, 2, _CHUNK), jnp.int32),
            pltpu.VMEM((_NB * _CHUNK,), jnp.float32),
            pltpu.VMEM((_NB * _CHUNK, _H), jnp.float32),
            pltpu.VMEM_SHARED((_ACC_ROWS, _H), jnp.float32),
            pltpu.SemaphoreType.DMA((2,)),
            pltpu.SemaphoreType.DMA((2,)),
            pltpu.SemaphoreType.DMA,
        ],
        compiler_params=pltpu.CompilerParams(use_tc_tiling_on_sc=False),
    )
    def spmm_kernel(ego_hbm, meta0_hbm, meta1_hbm, val_hbm, out_hbm,
                    meta_buf, val_buf, rows_buf, acc, sem_i, sem_g, sem_s):
        c = lax.axis_index("c")
        s = lax.axis_index("s")

        def fire_idx(g, k):
            cid = s * _NCHUNK + g

            @pl.when(c == 0)
            def _f0():
                pltpu.async_copy(
                    meta0_hbm.at[cid], meta_buf.at[k], sem_i.at[g % 2])

            @pl.when(c == 1)
            def _f1():
                pltpu.async_copy(
                    meta1_hbm.at[cid], meta_buf.at[k], sem_i.at[g % 2])

            pltpu.async_copy(
                val_hbm.at[cid], val_buf.at[pl.ds(k * _CHUNK, _CHUNK)],
                sem_i.at[g % 2])

        def wait_idx(g, k):
            pltpu.make_async_copy(
                meta0_hbm.at[0], meta_buf.at[k], sem_i.at[g % 2]).wait()
            pltpu.make_async_copy(
                val_hbm.at[0], val_buf.at[pl.ds(k * _CHUNK, _CHUNK)],
                sem_i.at[g % 2]).wait()

        def fire_gather(g, k):
            pltpu.async_copy(
                ego_hbm.at[meta_buf.at[k, 0]],
                rows_buf.at[pl.ds(k * _CHUNK, _CHUNK)], sem_g.at[g % 2])

        def wait_gather(g, k):
            pltpu.make_async_copy(
                ego_hbm.at[meta_buf.at[k, 0]],
                rows_buf.at[pl.ds(k * _CHUNK, _CHUNK)], sem_g.at[g % 2]).wait()

        def fire_scatter(k):
            pltpu.async_copy(
                rows_buf.at[pl.ds(k * _CHUNK, _CHUNK)],
                acc.at[meta_buf.at[k, 1]], sem_s, add=True)

        def wait_scatter(k):
            pltpu.make_async_copy(
                rows_buf.at[pl.ds(k * _CHUNK, _CHUNK)],
                acc.at[meta_buf.at[k, 1]], sem_s).wait()

        zro = jnp.zeros((_LANES,), jnp.float32)

        def zero_body(i, _):
            rows_buf[i // 2, pl.ds((i % 2) * _LANES, _LANES)] = zro
            return ()
        lax.fori_loop(0, _NB * _CHUNK * 2, zero_body, (), unroll=8)

        zrows = _NB * _CHUNK
        nfull = _ZSTRIPE // zrows
        zrem = _ZSTRIPE % zrows
        for j in range(nfull):
            pltpu.sync_copy(
                rows_buf, acc.at[pl.ds(s * _ZSTRIPE + j * zrows, zrows)])
        if zrem:
            pltpu.sync_copy(
                rows_buf.at[pl.ds(0, zrem)],
                acc.at[pl.ds(s * _ZSTRIPE + nfull * zrows, zrem)])
        plsc.subcore_barrier()

        fire_idx(0, 0)
        fire_idx(1, 1)
        wait_idx(0, 0)
        fire_gather(0, 0)

        def chunk_body(g, _):
            k = g % _NB
            kn = (g + 1) % _NB
            kn2 = (g + 2) % _NB

            @pl.when(g < _NCHUNK - 1)
            def _advance():
                wait_idx(g + 1, kn)
                fire_gather(g + 1, kn)

            wait_gather(g, k)

            dnums = lax.GatherDimensionNumbers(
                offset_dims=(), collapsed_slice_dims=(0,),
                start_index_map=(0,))

            @plsc.parallel_loop(0, _CHUNK // _LANES, 1, unroll=2)
            def mul_body(b):
                e0 = b * _LANES
                vv = val_buf[pl.ds(k * _CHUNK + e0, _LANES)]
                for l in range(_LANES):
                    bv = lax.gather(
                        vv, jnp.zeros((_LANES, 1), jnp.int32) + l, dnums,
                        (1,), mode=lax.GatherScatterMode.PROMISE_IN_BOUNDS)
                    r = k * _CHUNK + e0 + l
                    rows_buf[r, pl.ds(0, _LANES)] = (
                        rows_buf[r, pl.ds(0, _LANES)] * bv)
                    rows_buf[r, pl.ds(_LANES, _LANES)] = (
                        rows_buf[r, pl.ds(_LANES, _LANES)] * bv)

            @pl.when(g >= 1)
            def _drain_scatter():
                wait_scatter(kn2)

            @pl.when(g < _NCHUNK - 2)
            def _prefetch():
                fire_idx(g + 2, kn2)

            fire_scatter(k)
            return ()
        lax.fori_loop(0, _NCHUNK, chunk_body, ())
        wait_scatter((_NCHUNK - 1) % _NB)

        plsc.subcore_barrier()
        last = _N_NODES - (_NS - 1) * _OSTRIPE

        @pl.when(s < _NS - 1)
        def _copy_main():
            pltpu.sync_copy(
                acc.at[pl.ds(s * _OSTRIPE, _OSTRIPE)],
                out_hbm.at[pl.ds(c * _N_NODES + s * _OSTRIPE, _OSTRIPE)])

        @pl.when(s == _NS - 1)
        def _copy_last():
            pltpu.sync_copy(
                acc.at[pl.ds((_NS - 1) * _OSTRIPE, last)],
                out_hbm.at[pl.ds(c * _N_NODES + (_NS - 1) * _OSTRIPE, last)])

    return spmm_kernel(ego_stack, meta0, meta1, val2)


_R = 2000
_G = _N_NODES // _R


def _tc_dense(side_stack, ego_stack, acc, w_gc, b_gc, w_bi, b_bi, scale):
    def body(sl, sr, el, er, acc_in, wgc, bgc, wbi, bbi, ego_out, acc_out):
        s_full = jnp.concatenate([sl[...], sr[...]], axis=1)
        e_full = jnp.concatenate([el[...], er[...]], axis=1)
        sum_e = jnp.dot(s_full, wgc[...],
                        preferred_element_type=jnp.float32) + bgc[...]
        bi = jnp.dot(e_full * s_full, wbi[...],
                     preferred_element_type=jnp.float32) + bbi[...]
        x = sum_e + bi
        x = jnp.where(x >= 0, x, 0.2 * x)
        n = jnp.sqrt(jnp.sum(x * x, axis=1, keepdims=True))
        xn = x / jnp.maximum(n, 1e-12)
        acc_out[...] = (acc_in[...] + xn) * scale
        ego_out[0] = x[:, :_H]
        ego_out[1] = x[:, _H:]

    half = lambda off: pl.BlockSpec((_R, _H), lambda i: (i + off, 0))
    full_rows = pl.BlockSpec((_R, _D), lambda i: (i, 0))
    wspec = pl.BlockSpec((_D, _D), lambda i: (0, 0))
    bspec = pl.BlockSpec((1, _D), lambda i: (0, 0))
    ego3, acc_new = pl.pallas_call(
        body,
        grid=(_G,),
        in_specs=[half(0), half(_G), half(0), half(_G), full_rows,
                  wspec, bspec, wspec, bspec],
        out_specs=[pl.BlockSpec((2, _R, _H), lambda i: (0, i, 0)), full_rows],
        out_shape=[
            jax.ShapeDtypeStruct((2, _N_NODES, _H), jnp.float32),
            jax.ShapeDtypeStruct((_N_NODES, _D), jnp.float32),
        ],
    )(side_stack, side_stack, ego_stack, ego_stack, acc,
      w_gc, b_gc, w_bi, b_bi)
    return ego3, acc_new


def kernel(user_emb, item_emb, adj_val,
           W_gc_0, b_gc_0, W_bi_0, b_bi_0,
           W_gc_1, b_gc_1, W_bi_1, b_bi_1,
           W_gc_2, b_gc_2, W_bi_2, b_bi_2,
           adj_row, adj_col):
    ws_gc = [W_gc_0, W_gc_1, W_gc_2]
    bs_gc = [b_gc_0, b_gc_1, b_gc_2]
    ws_bi = [W_bi_0, W_bi_1, W_bi_2]
    bs_bi = [b_bi_0, b_bi_1, b_bi_2]

    ego_l = jnp.concatenate([user_emb[:, :_H], item_emb[:, :_H]], axis=0)
    ego_r = jnp.concatenate([user_emb[:, _H:], item_emb[:, _H:]], axis=0)
    ego_stack = jnp.concatenate([ego_l, ego_r], axis=0)

    pad = _EPAD - _E
    colp = jnp.concatenate([adj_col, jnp.zeros((pad,), jnp.int32)])
    rowp = jnp.concatenate([adj_row, jnp.zeros((pad,), jnp.int32)])
    ncid = _NS * _NCHUNK
    val2 = jnp.concatenate(
        [adj_val, jnp.zeros((pad,), jnp.float32)]).reshape(ncid, _CHUNK)
    meta0 = jnp.stack(
        [colp.reshape(ncid, _CHUNK), rowp.reshape(ncid, _CHUNK)], axis=1)
    meta1 = meta0.at[:, 0, :].add(_N_NODES)

    acc = jnp.concatenate([user_emb, item_emb], axis=0)
    n_layers = 3
    for k in range(n_layers):
        side_stack = _sc_spmm(ego_stack, meta0, meta1, val2)
        scale = 0.25 if k == n_layers - 1 else 1.0
        ego3, acc = _tc_dense(side_stack, ego_stack, acc,
                              ws_gc[k], bs_gc[k], ws_bi[k], bs_bi[k], scale)
        ego_stack = ego3.reshape(_NC * _N_NODES, _H)

    return acc[:_N_USERS], acc[_N_USERS:]


# speedup vs baseline: 2.4384x; 1.3974x over previous
import functools

import numpy as np

import jax
import jax.numpy as jnp
from jax import lax
from jax.experimental import pallas as pl
from jax.experimental.pallas import tpu as pltpu
from jax.experimental.pallas import tpu_sc as plsc

_N_USERS = 25000
_N_NODES = 50000
_E = 800000
_D = 64
_H = 32
_NC = 2
_NS = 16
_LANES = 16

_CHUNK = 256
_TILE_E = _E // _NS
_NCHUNK = _TILE_E // _CHUNK
_TAIL = _TILE_E - _NCHUNK * _CHUNK
_NB = 3
_NP = 51200
_ACC_ROWS = _NP
_ZSTRIPE = _ACC_ROWS // _NS
_OSTRIPE = _NP // _NS


def _sc_spmm(ego_stack, col, row, val):
    mesh = plsc.VectorSubcoreMesh(
        core_axis_name="c", subcore_axis_name="s",
        num_cores=_NC, num_subcores=_NS)

    @functools.partial(
        pl.kernel,
        out_type=jax.ShapeDtypeStruct((_NC * _NP, _H), jnp.float32),
        mesh=mesh,
        scratch_types=[
            pltpu.VMEM((_NB, _CHUNK), jnp.int32),
            pltpu.VMEM((_NB, _CHUNK), jnp.int32),
            pltpu.VMEM((_NB * _CHUNK,), jnp.float32),
            pltpu.VMEM((_NB * _CHUNK, _H), jnp.float32),
            pltpu.VMEM_SHARED((_ACC_ROWS, _H), jnp.float32),
            pltpu.SemaphoreType.DMA((2,)),
            pltpu.SemaphoreType.DMA((2,)),
            pltpu.SemaphoreType.DMA,
        ],
        compiler_params=pltpu.CompilerParams(use_tc_tiling_on_sc=False),
    )
    def spmm_kernel(ego_hbm, col_hbm, row_hbm, val_hbm, out_hbm,
                    col_buf, row_buf, val_buf, rows_buf, acc,
                    sem_i, sem_g, sem_s):
        c = lax.axis_index("c")
        s = lax.axis_index("s")

        def fire_idx(g, k):
            eb = s * _TILE_E + g * _CHUNK
            pltpu.async_copy(
                col_hbm.at[pl.ds(eb, _CHUNK)], col_buf.at[k], sem_i.at[g % 2])
            pltpu.async_copy(
                row_hbm.at[pl.ds(eb, _CHUNK)], row_buf.at[k], sem_i.at[g % 2])
            pltpu.async_copy(
                val_hbm.at[pl.ds(eb, _CHUNK)],
                val_buf.at[pl.ds(k * _CHUNK, _CHUNK)], sem_i.at[g % 2])

        def wait_idx(g, k):
            pltpu.make_async_copy(
                col_hbm.at[pl.ds(0, _CHUNK)], col_buf.at[k],
                sem_i.at[g % 2]).wait()
            pltpu.make_async_copy(
                row_hbm.at[pl.ds(0, _CHUNK)], row_buf.at[k],
                sem_i.at[g % 2]).wait()
            pltpu.make_async_copy(
                val_hbm.at[pl.ds(0, _CHUNK)],
                val_buf.at[pl.ds(k * _CHUNK, _CHUNK)], sem_i.at[g % 2]).wait()

        coffs = c * _NP

        def offset_cols(k, ngroups):
            @plsc.parallel_loop(0, ngroups, 1, unroll=4)
            def add_body(b):
                sl = pl.ds(b * _LANES, _LANES)
                col_buf[k, sl] = (
                    col_buf[k, sl] + (jnp.zeros((_LANES,), jnp.int32) + coffs))

        def fire_gather(g, k):
            pltpu.async_copy(
                ego_hbm.at[col_buf.at[k]],
                rows_buf.at[pl.ds(k * _CHUNK, _CHUNK)], sem_g.at[g % 2])

        def wait_gather(g, k):
            pltpu.make_async_copy(
                ego_hbm.at[col_buf.at[k]],
                rows_buf.at[pl.ds(k * _CHUNK, _CHUNK)], sem_g.at[g % 2]).wait()

        def fire_scatter(k):
            pltpu.async_copy(
                rows_buf.at[pl.ds(k * _CHUNK, _CHUNK)],
                acc.at[row_buf.at[k]], sem_s, add=True)

        def wait_scatter(k):
            pltpu.make_async_copy(
                rows_buf.at[pl.ds(k * _CHUNK, _CHUNK)],
                acc.at[row_buf.at[k]], sem_s).wait()

        dnums = lax.GatherDimensionNumbers(
            offset_dims=(), collapsed_slice_dims=(0,), start_index_map=(0,))

        def mul_groups(kbase, ngroups):
            @plsc.parallel_loop(0, ngroups, 1, unroll=4)
            def mul_body(b):
                e0 = b * _LANES
                vv = val_buf[pl.ds(kbase + e0, _LANES)]
                for l in range(_LANES):
                    bv = lax.gather(
                        vv, jnp.zeros((_LANES, 1), jnp.int32) + l, dnums,
                        (1,), mode=lax.GatherScatterMode.PROMISE_IN_BOUNDS)
                    r = kbase + e0 + l
                    rows_buf[r, pl.ds(0, _LANES)] = (
                        rows_buf[r, pl.ds(0, _LANES)] * bv)
                    rows_buf[r, pl.ds(_LANES, _LANES)] = (
                        rows_buf[r, pl.ds(_LANES, _LANES)] * bv)

        zro = jnp.zeros((_LANES,), jnp.float32)

        def zero_body(i, _):
            rows_buf[i // 2, pl.ds((i % 2) * _LANES, _LANES)] = zro
            return ()
        lax.fori_loop(0, _NB * _CHUNK * 2, zero_body, (), unroll=8)

        zrows = _NB * _CHUNK
        nfull = _ZSTRIPE // zrows
        zrem = _ZSTRIPE % zrows
        for j in range(nfull):
            pltpu.sync_copy(
                rows_buf, acc.at[pl.ds(s * _ZSTRIPE + j * zrows, zrows)])
        if zrem:
            pltpu.sync_copy(
                rows_buf.at[pl.ds(0, zrem)],
                acc.at[pl.ds(s * _ZSTRIPE + nfull * zrows, zrem)])
        plsc.subcore_barrier()

        fire_idx(0, 0)
        fire_idx(1, 1)
        wait_idx(0, 0)
        offset_cols(0, _CHUNK // _LANES)
        fire_gather(0, 0)

        def chunk_body(g, _):
            k = g % _NB
            kn = (g + 1) % _NB
            kn2 = (g + 2) % _NB

            @pl.when(g < _NCHUNK - 1)
            def _advance():
                wait_idx(g + 1, kn)
                offset_cols(kn, _CHUNK // _LANES)
                fire_gather(g + 1, kn)

            wait_gather(g, k)
            mul_groups(k * _CHUNK, _CHUNK // _LANES)

            @pl.when(g >= 1)
            def _drain_scatter():
                wait_scatter(kn2)

            @pl.when(g < _NCHUNK - 2)
            def _prefetch():
                fire_idx(g + 2, kn2)

            fire_scatter(k)
            return ()
        lax.fori_loop(0, _NCHUNK, chunk_body, ())
        wait_scatter((_NCHUNK - 1) % _NB)

        tb = s * _TILE_E + _NCHUNK * _CHUNK
        pltpu.sync_copy(col_hbm.at[pl.ds(tb, _TAIL)],
                        col_buf.at[0, pl.ds(0, _TAIL)])
        pltpu.sync_copy(row_hbm.at[pl.ds(tb, _TAIL)],
                        row_buf.at[0, pl.ds(0, _TAIL)])
        pltpu.sync_copy(val_hbm.at[pl.ds(tb, _TAIL)],
                        val_buf.at[pl.ds(0, _TAIL)])

        offset_cols(0, _TAIL // _LANES)
        pltpu.sync_copy(ego_hbm.at[col_buf.at[0, pl.ds(0, _TAIL)]],
                        rows_buf.at[pl.ds(0, _TAIL)])

        mul_groups(0, _TAIL // _LANES)
        pltpu.sync_copy(rows_buf.at[pl.ds(0, _TAIL)],
                        acc.at[row_buf.at[0, pl.ds(0, _TAIL)]], add=True)

        plsc.subcore_barrier()
        pltpu.sync_copy(
            acc.at[pl.ds(s * _OSTRIPE, _OSTRIPE)],
            out_hbm.at[pl.ds(c * _NP + s * _OSTRIPE, _OSTRIPE)])

    return spmm_kernel(ego_stack, col, row, val)


_R = 5120
_G = 51200 // _R


_RU = 1000


def _tc_prep(user_emb, item_emb):
    nbu = 25000 // _RU

    def body(u, it, ego_out, acc_out):
        t = pl.program_id(0)
        x = jnp.where(t == 0, u[...], it[...])
        acc_out[...] = x
        ego_out[0] = x[:, :_H]
        ego_out[1] = x[:, _H:]

    tab = pl.BlockSpec((_RU, _D), lambda t, i: (i, 0))
    off = lambda t, i: (t * nbu + i, 0)
    return pl.pallas_call(
        body,
        grid=(2, nbu),
        in_specs=[tab, tab],
        out_specs=[pl.BlockSpec((2, _RU, _H), lambda t, i: (0, t * nbu + i, 0)),
                   pl.BlockSpec((_RU, _D), off)],
        out_shape=[
            jax.ShapeDtypeStruct((2, _NP, _H), jnp.float32),
            jax.ShapeDtypeStruct((_NP, _D), jnp.float32),
        ],
    )(user_emb, item_emb)


_KMAT = np.zeros((256, 4), np.float32)
_BMAT = np.zeros((4, 256), np.float32)
_PLMAT = np.zeros((256, 128), np.float32)
_PRMAT = np.zeros((256, 128), np.float32)
for _u in range(4):
    for _d in range(64):
        _KMAT[64 * _u + _d, _u] = 1.0
        _BMAT[_u, 64 * _u + _d] = 1.0
    for _c in range(32):
        _PLMAT[64 * _u + _c, 32 * _u + _c] = 1.0
        _PRMAT[64 * _u + 32 + _c, 32 * _u + _c] = 1.0


def _tc_dense(side_pk, ego_pk, acc_pk, w2_gc, b2_gc, w2_bi, b2_bi, scale):
    def body(sl, sr, el, er, acc_in, wgc, bgc, wbi, bbi,
             km, bm, plm, prm, ego_out, acc_out):
        s2 = jnp.concatenate([sl[...], sr[...]], axis=1)
        e2 = jnp.concatenate([el[...], er[...]], axis=1)
        sum_e = jnp.dot(s2, wgc[...],
                        preferred_element_type=jnp.float32) + bgc[...]
        bi = jnp.dot(e2 * s2, wbi[...],
                     preferred_element_type=jnp.float32) + bbi[...]
        x = sum_e + bi
        x = jnp.where(x >= 0, x, 0.2 * x)
        n2 = jnp.dot(x * x, km[...], preferred_element_type=jnp.float32)
        nb = jnp.dot(n2, bm[...], preferred_element_type=jnp.float32)
        xn = x / jnp.maximum(jnp.sqrt(nb), 1e-12)
        acc_out[...] = (acc_in[...] + xn) * scale
        ego_out[0] = jnp.dot(x, plm[...], preferred_element_type=jnp.float32)
        ego_out[1] = jnp.dot(x, prm[...], preferred_element_type=jnp.float32)

    rq = _R // 4
    pk_lo = pl.BlockSpec((rq, 128), lambda i: (i, 0))
    pk_hi = pl.BlockSpec((rq, 128), lambda i: (i + _G, 0))
    acc_spec = pl.BlockSpec((rq, 256), lambda i: (i, 0))
    cspec = lambda shape: pl.BlockSpec(shape, lambda i: tuple(0 for _ in shape))
    return pl.pallas_call(
        body,
        grid=(_G,),
        in_specs=[pk_lo, pk_hi, pk_lo, pk_hi, acc_spec,
                  cspec((256, 256)), cspec((1, 256)),
                  cspec((256, 256)), cspec((1, 256)),
                  cspec((256, 4)), cspec((4, 256)),
                  cspec((256, 128)), cspec((256, 128))],
        out_specs=[pl.BlockSpec((2, rq, 128), lambda i: (0, i, 0)), acc_spec],
        out_shape=[
            jax.ShapeDtypeStruct((2, _NP // 4, 128), jnp.float32),
            jax.ShapeDtypeStruct((_NP // 4, 256), jnp.float32),
        ],
    )(side_pk, side_pk, ego_pk, ego_pk, acc_pk,
      w2_gc, b2_gc, w2_bi, b2_bi, _KMAT, _BMAT, _PLMAT, _PRMAT)


def kernel(user_emb, item_emb, adj_val,
           W_gc_0, b_gc_0, W_bi_0, b_bi_0,
           W_gc_1, b_gc_1, W_bi_1, b_bi_1,
           W_gc_2, b_gc_2, W_bi_2, b_bi_2,
           adj_row, adj_col):
    ws_gc = [W_gc_0, W_gc_1, W_gc_2]
    bs_gc = [b_gc_0, b_gc_1, b_gc_2]
    ws_bi = [W_bi_0, W_bi_1, W_bi_2]
    bs_bi = [b_bi_0, b_bi_1, b_bi_2]

    ego3, acc0 = _tc_prep(user_emb, item_emb)
    ego_stack = ego3.reshape(_NC * _NP, _H)
    ego_pk = ego3.reshape(_NC * _NP // 4, 128)
    acc = acc0.reshape(_NP // 4, 256)
    eye4 = jnp.eye(4, dtype=jnp.float32)
    n_layers = 3
    for k in range(n_layers):
        w2_gc = jnp.concatenate([jnp.kron(eye4, ws_gc[k][:_H, :]),
                                 jnp.kron(eye4, ws_gc[k][_H:, :])], axis=0)
        w2_bi = jnp.concatenate([jnp.kron(eye4, ws_bi[k][:_H, :]),
                                 jnp.kron(eye4, ws_bi[k][_H:, :])], axis=0)
        b2_gc = jnp.tile(bs_gc[k], (1, 4))
        b2_bi = jnp.tile(bs_bi[k], (1, 4))
        side_stack = _sc_spmm(ego_stack, adj_col, adj_row, adj_val)
        side_pk = side_stack.reshape(_NC * _NP // 4, 128)
        scale = 0.25 if k == n_layers - 1 else 1.0
        ego3, acc = _tc_dense(side_pk, ego_pk, acc,
                              w2_gc, b2_gc, w2_bi, b2_bi, scale)
        ego_stack = ego3.reshape(_NC * _NP, _H)
        ego_pk = ego3.reshape(_NC * _NP // 4, 128)
    accf = acc.reshape(_NP, _D)
    return accf[:_N_USERS], accf[_N_USERS:_N_NODES]
